# baseline (speedup 1.0000x reference)
import functools
import math

import jax
import jax.numpy as jnp
from jax import lax
from jax.experimental import pallas as pl
from jax.experimental.pallas import tpu as pltpu
from jax.experimental.pallas import tpu_sc as plsc

_info = plsc.get_sparse_core_info()
_NC, _NS, _L = _info.num_cores, _info.num_subcores, _info.num_lanes
_NW = _NC * _NS
_SB = 128


def _emb_kernel(n_tok: int, dim: int, units_per_w: int,
                table2_hbm, tok_hbm, out_hbm,
                idx_raw, idx2, pcol, rows, outt, gsems, osems):
    wid = lax.axis_index("s") * _NC + lax.axis_index("c")
    base_u = wid * units_per_w
    lane0 = lax.iota(jnp.int32, _L)

    def prep_and_fire(k, b):
        u = base_u + k
        t = u >> 7
        sb = u & 127
        pltpu.sync_copy(tok_hbm.at[t, pl.ds(sb * _SB, _SB)], idx_raw)
        for j in range(_SB // _L):
            v = idx_raw[pl.ds(j * _L, _L)]
            idx2[b][pl.ds(j * _L, _L)] = v >> 1
            pcol[b][pl.ds(j * _L, _L)] = (v & 1) << 6
        pltpu.async_copy(table2_hbm.at[idx2[b]], rows[b], gsems[b])

    prep_and_fire(0, 0)
    prep_and_fire(1, 1)

    def pair_body(i, _):
        for b in range(2):
            k = i * 2 + b
            u = base_u + k
            t = u >> 7
            sb = u & 127
            pltpu.make_async_copy(table2_hbm.at[idx2[b]], rows[b],
                                  gsems[b]).wait()

            @pl.when(i > 0)
            def _():
                pltpu.make_async_copy(
                    outt[b], out_hbm.at[t, :, pl.ds(sb * _SB, _SB)],
                    osems[b]).wait()

            for lg in range(_SB // _L):
                lanes = lane0 + lg * _L
                pv = pcol[b][pl.ds(lg * _L, _L)]

                def dbody(d, _, lanes=lanes, pv=pv, b=b, lg=lg):
                    cols = pv + d
                    v = plsc.load_gather(rows[b], [lanes, cols])
                    outt[b][d, pl.ds(lg * _L, _L)] = v * 8.0
                    return 0

                lax.fori_loop(0, dim, dbody, 0, unroll=8)

            pltpu.async_copy(outt[b],
                             out_hbm.at[t, :, pl.ds(sb * _SB, _SB)],
                             osems[b])

            @pl.when(k + 2 < units_per_w)
            def _():
                prep_and_fire(k + 2, b)

        return 0

    lax.fori_loop(0, units_per_w // 2, pair_body, 0)

    for b in range(2):
        k = units_per_w - 2 + b
        u = base_u + k
        t = u >> 7
        sb = u & 127
        pltpu.make_async_copy(outt[b],
                              out_hbm.at[t, :, pl.ds(sb * _SB, _SB)],
                              osems[b]).wait()


def kernel(tokens, table):
    n_seq, n_tok = tokens.shape
    vocab, dim = table.shape
    units = n_tok * (n_seq // _SB)
    units_per_w = units // _NW

    table2 = table.reshape(vocab // 2, 2 * dim)
    tok_t = tokens.T.astype(jnp.int32)

    mesh = plsc.VectorSubcoreMesh(core_axis_name="c", subcore_axis_name="s")
    k = pl.kernel(
        functools.partial(_emb_kernel, n_tok, dim, units_per_w),
        mesh=mesh,
        out_type=jax.ShapeDtypeStruct((n_tok, dim, n_seq), jnp.float32),
        scratch_types=[
            pltpu.VMEM((_SB,), jnp.int32),
            [pltpu.VMEM((_SB,), jnp.int32) for _ in range(2)],
            [pltpu.VMEM((_SB,), jnp.int32) for _ in range(2)],
            [pltpu.VMEM((_SB, 2 * dim), jnp.float32) for _ in range(2)],
            [pltpu.VMEM((dim, _SB), jnp.float32) for _ in range(2)],
            [pltpu.SemaphoreType.DMA for _ in range(2)],
            [pltpu.SemaphoreType.DMA for _ in range(2)],
        ],
        compiler_params=pltpu.CompilerParams(use_tc_tiling_on_sc=True,
                                             needs_layout_passes=False),
    )
    out3d = k(table2, tok_t)
    return out3d.transpose(2, 0, 1)


# speedup vs baseline: 1.3048x; 1.3048x over previous
import functools
import math

import jax
import jax.numpy as jnp
from jax import lax
from jax.experimental import pallas as pl
from jax.experimental.pallas import tpu as pltpu
from jax.experimental.pallas import tpu_sc as plsc

_info = plsc.get_sparse_core_info()
_NC, _NS, _L = _info.num_cores, _info.num_subcores, _info.num_lanes
_NW = _NC * _NS
_SB = 128


def _emb_kernel(n_tok: int, dim: int, units_per_w: int,
                table2_hbm, tok_hbm, out_hbm,
                idx_raw, idx2, pcol, rows, outt, gsems, osems):
    wid = lax.axis_index("s") * _NC + lax.axis_index("c")
    base_u = wid * units_per_w
    lane0 = lax.iota(jnp.int32, _L)

    def prep_and_fire(k, b):
        u = base_u + k
        t = u >> 7
        sb = u & 127
        pltpu.sync_copy(tok_hbm.at[t, pl.ds(sb * _SB, _SB)], idx_raw)
        for j in range(_SB // _L):
            v = idx_raw[pl.ds(j * _L, _L)]
            idx2[b][pl.ds(j * _L, _L)] = v >> 1
            pcol[b][pl.ds(j * _L, _L)] = (v & 1) << 6
        pltpu.async_copy(table2_hbm.at[idx2[b]], rows[b], gsems[b])

    prep_and_fire(0, 0)
    prep_and_fire(1, 1)

    def pair_body(i, _):
        for b in range(2):
            k = i * 2 + b
            u = base_u + k
            t = u >> 7
            sb = u & 127
            pltpu.make_async_copy(table2_hbm.at[idx2[b]], rows[b],
                                  gsems[b]).wait()

            @pl.when(i > 0)
            def _():
                pltpu.make_async_copy(
                    outt[b], out_hbm.at[t, :, pl.ds(sb * _SB, _SB)],
                    osems[b]).wait()

            for lg in range(_SB // _L):
                lanes = lane0 + lg * _L
                pv = pcol[b][pl.ds(lg * _L, _L)]

                @plsc.parallel_loop(0, dim, unroll=8)
                def dbody(d, lanes=lanes, pv=pv, b=b, lg=lg):
                    cols = pv + d
                    v = plsc.load_gather(rows[b], [lanes, cols])
                    outt[b][d, pl.ds(lg * _L, _L)] = v * 8.0

            pltpu.async_copy(outt[b],
                             out_hbm.at[t, :, pl.ds(sb * _SB, _SB)],
                             osems[b])

            @pl.when(k + 2 < units_per_w)
            def _():
                prep_and_fire(k + 2, b)

        return 0

    lax.fori_loop(0, units_per_w // 2, pair_body, 0)

    for b in range(2):
        k = units_per_w - 2 + b
        u = base_u + k
        t = u >> 7
        sb = u & 127
        pltpu.make_async_copy(outt[b],
                              out_hbm.at[t, :, pl.ds(sb * _SB, _SB)],
                              osems[b]).wait()


def kernel(tokens, table):
    n_seq, n_tok = tokens.shape
    vocab, dim = table.shape
    units = n_tok * (n_seq // _SB)
    units_per_w = units // _NW

    table2 = table.reshape(vocab // 2, 2 * dim)
    tok_t = tokens.T.astype(jnp.int32)

    mesh = plsc.VectorSubcoreMesh(core_axis_name="c", subcore_axis_name="s")
    k = pl.kernel(
        functools.partial(_emb_kernel, n_tok, dim, units_per_w),
        mesh=mesh,
        out_type=jax.ShapeDtypeStruct((n_tok, dim, n_seq), jnp.float32),
        scratch_types=[
            pltpu.VMEM((_SB,), jnp.int32),
            [pltpu.VMEM((_SB,), jnp.int32) for _ in range(2)],
            [pltpu.VMEM((_SB,), jnp.int32) for _ in range(2)],
            [pltpu.VMEM((_SB, 2 * dim), jnp.float32) for _ in range(2)],
            [pltpu.VMEM((dim, _SB), jnp.float32) for _ in range(2)],
            [pltpu.SemaphoreType.DMA for _ in range(2)],
            [pltpu.SemaphoreType.DMA for _ in range(2)],
        ],
        compiler_params=pltpu.CompilerParams(use_tc_tiling_on_sc=True,
                                             needs_layout_passes=False),
    )
    out3d = k(table2, tok_t)
    return out3d.transpose(2, 0, 1)


# speedup vs baseline: 1.3550x; 1.0384x over previous
import functools

import jax
import jax.numpy as jnp
from jax import lax
from jax.experimental import pallas as pl
from jax.experimental.pallas import tpu as pltpu
from jax.experimental.pallas import tpu_sc as plsc

_info = plsc.get_sparse_core_info()
_NC, _NS, _L = _info.num_cores, _info.num_subcores, _info.num_lanes
_NW = _NC * _NS
_SB = 128


def _emb_kernel(n_tok: int, dim: int, blocks_per_w: int,
                table2_hbm, tok_hbm, out_hbm,
                idx_all, idx2, pcol, rows, outt, gsems, osems):
    wid = lax.axis_index("s") * _NC + lax.axis_index("c")
    seq0 = wid * (blocks_per_w * _SB)
    units_per_w = n_tok * blocks_per_w
    lane0 = lax.iota(jnp.int32, _L)

    pltpu.sync_copy(tok_hbm.at[:, pl.ds(seq0, blocks_per_w * _SB)], idx_all)

    def unit_coords(k):
        t = k // blocks_per_w
        sbl = k % blocks_per_w
        return t, sbl

    def prep_and_fire(k, b):
        t, sbl = unit_coords(k)
        for j in range(_SB // _L):
            v = idx_all[t, pl.ds(sbl * _SB + j * _L, _L)]
            idx2[b][pl.ds(j * _L, _L)] = v >> 1
            pcol[b][pl.ds(j * _L, _L)] = (v & 1) << 6
        pltpu.async_copy(table2_hbm.at[idx2[b]], rows[b], gsems[b])

    prep_and_fire(0, 0)
    prep_and_fire(1, 1)

    def pair_body(i, _):
        for b in range(2):
            k = i * 2 + b
            t, sbl = unit_coords(k)
            dst = out_hbm.at[t, :, pl.ds(seq0 + sbl * _SB, _SB)]
            pltpu.make_async_copy(table2_hbm.at[idx2[b]], rows[b],
                                  gsems[b]).wait()

            @pl.when(i > 0)
            def _():
                pltpu.make_async_copy(outt[b], dst, osems[b]).wait()

            for lg in range(_SB // _L):
                lanes = lane0 + lg * _L
                pv = pcol[b][pl.ds(lg * _L, _L)]

                @plsc.parallel_loop(0, dim, unroll=8)
                def dbody(d, lanes=lanes, pv=pv, b=b, lg=lg):
                    cols = pv + d
                    v = plsc.load_gather(rows[b], [lanes, cols])
                    outt[b][d, pl.ds(lg * _L, _L)] = v * 8.0

            pltpu.async_copy(outt[b], dst, osems[b])

            @pl.when(k + 2 < units_per_w)
            def _():
                prep_and_fire(k + 2, b)

        return 0

    lax.fori_loop(0, units_per_w // 2, pair_body, 0)

    for b in range(2):
        k = units_per_w - 2 + b
        t, sbl = unit_coords(k)
        dst = out_hbm.at[t, :, pl.ds(seq0 + sbl * _SB, _SB)]
        pltpu.make_async_copy(outt[b], dst, osems[b]).wait()


def kernel(tokens, table):
    n_seq, n_tok = tokens.shape
    vocab, dim = table.shape
    blocks_per_w = n_seq // _SB // _NW

    table2 = table.reshape(vocab // 2, 2 * dim)
    tok_t = tokens.T.astype(jnp.int32)

    mesh = plsc.VectorSubcoreMesh(core_axis_name="c", subcore_axis_name="s")
    k = pl.kernel(
        functools.partial(_emb_kernel, n_tok, dim, blocks_per_w),
        mesh=mesh,
        out_type=jax.ShapeDtypeStruct((n_tok, dim, n_seq), jnp.float32),
        scratch_types=[
            pltpu.VMEM((n_tok, blocks_per_w * _SB), jnp.int32),
            [pltpu.VMEM((_SB,), jnp.int32) for _ in range(2)],
            [pltpu.VMEM((_SB,), jnp.int32) for _ in range(2)],
            [pltpu.VMEM((_SB, 2 * dim), jnp.float32) for _ in range(2)],
            [pltpu.VMEM((dim, _SB), jnp.float32) for _ in range(2)],
            [pltpu.SemaphoreType.DMA for _ in range(2)],
            [pltpu.SemaphoreType.DMA for _ in range(2)],
        ],
        compiler_params=pltpu.CompilerParams(use_tc_tiling_on_sc=True,
                                             needs_layout_passes=False),
    )
    out3d = k(table2, tok_t)
    return out3d.transpose(2, 0, 1)
